# Initial kernel scaffold; baseline (speedup 1.0000x reference)
import jax
import jax.numpy as jnp
from jax.experimental import pallas as pl


def kernel(x, W):
    raise NotImplementedError("write your pallas kernel here")


# speedup vs baseline: 4.6206x; 4.6206x over previous
import functools

import jax
import jax.numpy as jnp
from jax.experimental import pallas as pl
from jax.experimental.pallas import tpu as pltpu

N_TOK = 32768
D = 4096
E = 64
TOPK = 8
BT = 512


def _router_block(x_ref, wt_ref, mask_ref, scores_ref, idx_ref, sums_ref, aux_ref):
    i = pl.program_id(0)
    ni = pl.num_programs(0)

    logits = jnp.dot(x_ref[...], wt_ref[...], preferred_element_type=jnp.float32)
    m = jnp.max(logits, axis=1, keepdims=True)
    e = jnp.exp(logits - m)
    s = e / jnp.sum(e, axis=1, keepdims=True)
    scores_ref[...] = s

    lane = jax.lax.broadcasted_iota(jnp.int32, (BT, E), 1)
    work = s
    sel_acc = jnp.zeros((BT, E), dtype=jnp.float32)
    idx_cols = []
    for _ in range(TOPK):
        cur = jnp.max(work, axis=1, keepdims=True)
        idx = jnp.min(jnp.where(work == cur, lane, E), axis=1, keepdims=True)
        sel = lane == idx
        sel_acc = jnp.where(sel, 1.0, sel_acc)
        idx_cols.append(idx)
        work = jnp.where(sel, -jnp.inf, work)
    mask_ref[...] = sel_acc
    idx_ref[...] = jnp.concatenate(idx_cols, axis=1)

    imp = jnp.sum(s, axis=0)
    load = jnp.sum(sel_acc, axis=0)
    partial = jnp.concatenate(
        [imp[None, :], load[None, :], jnp.zeros((6, E), jnp.float32)], axis=0)

    @pl.when(i == 0)
    def _():
        sums_ref[...] = partial

    @pl.when(i > 0)
    def _():
        sums_ref[...] += partial

    @pl.when(i == ni - 1)
    def _():
        tot = sums_ref[...]
        aux = jnp.sum(tot[0:1, :] * tot[1:2, :]) * (E / (N_TOK * N_TOK))
        aux_ref[0, 0] = aux


@jax.jit
def _router(x, wt):
    grid = (N_TOK // BT,)
    mask, scores, idx, _sums, aux = pl.pallas_call(
        _router_block,
        grid=grid,
        in_specs=[
            pl.BlockSpec((BT, D), lambda i: (i, 0)),
            pl.BlockSpec((D, E), lambda i: (0, 0)),
        ],
        out_specs=[
            pl.BlockSpec((BT, E), lambda i: (i, 0)),
            pl.BlockSpec((BT, E), lambda i: (i, 0)),
            pl.BlockSpec((BT, TOPK), lambda i: (i, 0)),
            pl.BlockSpec((8, E), lambda i: (0, 0)),
            pl.BlockSpec(memory_space=pltpu.SMEM),
        ],
        out_shape=[
            jax.ShapeDtypeStruct((N_TOK, E), jnp.float32),
            jax.ShapeDtypeStruct((N_TOK, E), jnp.float32),
            jax.ShapeDtypeStruct((N_TOK, TOPK), jnp.int32),
            jax.ShapeDtypeStruct((8, E), jnp.float32),
            jax.ShapeDtypeStruct((1, 1), jnp.float32),
        ],
        compiler_params=pltpu.CompilerParams(
            dimension_semantics=("arbitrary",),
        ),
    )(x, wt)
    return mask, scores, aux[0, 0], idx


def kernel(x, W):
    return _router(x, W.T)
